# baseline (speedup 1.0000x reference)
import functools

import jax
import jax.numpy as jnp
from jax import lax
from jax.experimental import pallas as pl
from jax.experimental.pallas import tpu as pltpu
from jax.experimental.pallas import tpu_sc as plsc

_NC = 2
_NS = 16
_NW = _NC * _NS
_D = 80
_DP = 128
_C = 128
_NSLOT = 5


@functools.partial(jax.jit, static_argnames=("B",))
def _sc_embed(idx2d, table, B):
    b_per_w = B // _NW
    n_chunks = b_per_w // _C
    assert b_per_w % _C == 0 and n_chunks % _NSLOT == 0

    mesh = plsc.VectorSubcoreMesh(core_axis_name="c", subcore_axis_name="s")

    @functools.partial(
        pl.kernel,
        mesh=mesh,
        out_type=jax.ShapeDtypeStruct((B, _DP), jnp.float32),
        scratch_types=[
            pltpu.VMEM((_NSLOT, 1, _C), jnp.int32),
            pltpu.VMEM((_NSLOT, _C, _DP), jnp.float32),
        ]
        + [pltpu.SemaphoreType.DMA] * (2 * _NSLOT),
    )
    def k(idx_hbm, table_hbm, out_hbm, idx_v, rows_v, *sems):
        sg = sems[:_NSLOT]
        ss = sems[_NSLOT:]
        wid = lax.axis_index("s") * _NC + lax.axis_index("c")
        wrow0 = wid * n_chunks
        wbase = wid * b_per_w
        off = wid * 21
        ngroups = n_chunks // _NSLOT

        def stage_idx(g, slot):
            pltpu.sync_copy(idx_hbm.at[pl.ds(wrow0 + g, 1)], idx_v.at[slot])
            for q in range(_C // 16):
                sl = idx_v.at[slot, 0][pl.ds(q * 16, 16)]
                idx_v.at[slot, 0][pl.ds(q * 16, 16)] = sl + off

        def drain_scatter(slot):
            pltpu.make_async_copy(
                rows_v.at[slot], out_hbm.at[pl.ds(wbase, _C)], ss[slot]
            ).wait()

        def body(p, carry):
            g0 = _NSLOT * p
            gcps = []
            for s in range(_NSLOT):

                @pl.when(p >= 1)
                def _(s=s):
                    drain_scatter(s)

                stage_idx(g0 + s, s)
                gcps.append(
                    pltpu.async_copy(
                        table_hbm.at[idx_v.at[s, 0]], rows_v.at[s], sg[s]
                    )
                )
            for s in range(_NSLOT):
                gcps[s].wait()
                pltpu.async_copy(
                    rows_v.at[s],
                    out_hbm.at[pl.ds(wbase + (g0 + s) * _C, _C)],
                    ss[s],
                )
            return carry

        lax.fori_loop(0, ngroups, body, 0)
        for s in range(_NSLOT):
            drain_scatter(s)

    return k(idx2d, table)


def kernel(residue_types, table):
    S, T = residue_types.shape
    B = S * T
    idx2d = residue_types.reshape(B // 128, 128)
    table_rep = jnp.tile(jnp.pad(table, ((0, 0), (0, _DP - _D))), (_NW, 1))
    out = _sc_embed(idx2d, table_rep, B)
    return out[:, :_D].reshape(S, T, _D)
